# baseline (speedup 1.0000x reference)
import dataclasses
import functools

import jax
import jax.numpy as jnp
from jax import lax
from jax.experimental import pallas as pl
from jax.experimental.pallas import tpu as pltpu
from jax.experimental.pallas import tpu_sc as plsc

T = 2048
H = 1024
F = 4096
E = 8
K = 2

NENT = T * K
CHUNK = 256
NCHUNK = NENT // CHUNK + E
NSLOT = NCHUNK * CHUNK
NCPAD = 32

NWORK = 32
GW = 64


def _router_body(x_ref, wr_ref, pos_ref, prob_ref, expv_ref, valid_ref):
    logits = jnp.dot(x_ref[...], wr_ref[...], preferred_element_type=jnp.float32)
    eidx = lax.broadcasted_iota(jnp.int32, (T, E), 1)
    m1 = jnp.max(logits, axis=1, keepdims=True)
    i1 = jnp.min(jnp.where(logits == m1, eidx, E), axis=1, keepdims=True)
    l2 = jnp.where(eidx == i1, -jnp.inf, logits)
    m2 = jnp.max(l2, axis=1, keepdims=True)
    i2 = jnp.min(jnp.where(l2 == m2, eidx, E), axis=1, keepdims=True)
    e2 = jnp.exp(m2 - m1)
    p1 = 1.0 / (1.0 + e2)
    p2 = e2 / (1.0 + e2)

    a0 = jnp.where(eidx == i1, 1.0, 0.0)
    a1 = jnp.where(eidx == i2, 1.0, 0.0)
    b = a0 + a1

    r_iota = lax.broadcasted_iota(jnp.int32, (T, T), 0)
    c_iota = lax.broadcasted_iota(jnp.int32, (T, T), 1)
    ltri = jnp.where(r_iota > c_iota, 1.0, 0.0).astype(jnp.bfloat16)
    s = jnp.dot(ltri, b.astype(jnp.bfloat16), preferred_element_type=jnp.float32)

    counts = jnp.sum(b, axis=0, keepdims=True)
    nch = jnp.floor((counts + (CHUNK - 1)) * (1.0 / CHUNK))
    ru = lax.broadcasted_iota(jnp.int32, (E, E), 0)
    cu = lax.broadcasted_iota(jnp.int32, (E, E), 1)
    utri = jnp.where(ru < cu, 1.0, 0.0).astype(jnp.bfloat16)
    choff = jnp.dot(nch.astype(jnp.bfloat16), utri,
                    preferred_element_type=jnp.float32)
    slotbase = choff * float(CHUNK)

    base_bc = jnp.broadcast_to(slotbase, (T, E))
    pos0 = jnp.sum(a0 * (base_bc + s), axis=1, keepdims=True)
    pos1 = jnp.sum(a1 * (base_bc + s), axis=1, keepdims=True)

    pos_ref[...] = jnp.concatenate([pos0, pos1], axis=1).astype(jnp.int32)
    prob_ref[...] = jnp.concatenate([p1, p2], axis=1)

    total = jnp.sum(nch, axis=1, keepdims=True)
    cidx = lax.broadcasted_iota(jnp.int32, (NCPAD, E), 0).astype(jnp.float32)
    off_bc = jnp.broadcast_to(choff, (NCPAD, E))
    expv = jnp.sum(jnp.where(cidx >= off_bc, 1.0, 0.0), axis=1,
                   keepdims=True) - 1.0
    expv = jnp.clip(expv, 0.0, float(E - 1))
    expv_ref[...] = expv.astype(jnp.int32)
    cidx1 = lax.broadcasted_iota(jnp.int32, (NCPAD, 1), 0).astype(jnp.float32)
    valid_ref[...] = (cidx1 < jnp.broadcast_to(total, (NCPAD, 1))).astype(jnp.int32)


def _router(x, Wr):
    return pl.pallas_call(
        _router_body,
        out_shape=(
            jax.ShapeDtypeStruct((T, K), jnp.int32),
            jax.ShapeDtypeStruct((T, K), jnp.float32),
            jax.ShapeDtypeStruct((NCPAD, 1), jnp.int32),
            jax.ShapeDtypeStruct((NCPAD, 1), jnp.int32),
        ),
    )(x, Wr)


def _sc_mesh():
    return plsc.VectorSubcoreMesh(core_axis_name="c", subcore_axis_name="s")


def _sc_compiler_params():
    cp = pltpu.CompilerParams()
    if "needs_layout_passes" in pltpu.CompilerParams.__dataclass_fields__:
        cp = dataclasses.replace(cp, needs_layout_passes=False)
    return cp


def _build_gidx(pos_flat, zeros_slot):
    @functools.partial(
        pl.kernel,
        mesh=_sc_mesh(),
        out_type=jax.ShapeDtypeStruct((NSLOT,), jnp.int32),
        compiler_params=_sc_compiler_params(),
        scratch_types=[
            pltpu.VMEM((NSLOT,), jnp.int32),
            pltpu.VMEM((NENT,), jnp.int32),
        ],
    )
    def k(pos_hbm, zeros_hbm, gidx_hbm, gidx_v, pos_v):
        wid = lax.axis_index("s") * 2 + lax.axis_index("c")

        @pl.when(wid == 0)
        def _():
            pltpu.sync_copy(zeros_hbm, gidx_v)
            pltpu.sync_copy(pos_hbm, pos_v)

            @pl.loop(0, NENT // 16)
            def _(i):
                idx = pos_v[pl.ds(i * 16, 16)]
                vals = lax.shift_right_logical(
                    lax.iota(jnp.int32, 16) + i * 16, 1)
                plsc.store_scatter(gidx_v, [idx], vals)

            pltpu.sync_copy(gidx_v, gidx_hbm)

    return k(pos_flat, zeros_slot)


def _sc_gather(data, idx_flat, nrows):
    w = data.shape[1]
    nwin = nrows // (NWORK * GW)

    @functools.partial(
        pl.kernel,
        mesh=_sc_mesh(),
        out_type=jax.ShapeDtypeStruct((nrows, w), jnp.int32),
        scratch_types=(
            [pltpu.VMEM((GW,), jnp.int32) for _ in range(nwin)]
            + [pltpu.VMEM((GW, w), jnp.int32) for _ in range(nwin)]
            + [pltpu.SemaphoreType.DMA for _ in range(2 * nwin)]
        ),
    )
    def k(data_hbm, idx_hbm, out_hbm, *refs):
        idx_vs = refs[:nwin]
        row_vs = refs[nwin:2 * nwin]
        sems = refs[2 * nwin:]
        wid = lax.axis_index("s") * 2 + lax.axis_index("c")

        for j in range(nwin):
            pltpu.sync_copy(idx_hbm.at[pl.ds((wid * nwin + j) * GW, GW)],
                            idx_vs[j])
        gathers = [pltpu.async_copy(data_hbm.at[idx_vs[j]], row_vs[j], sems[j])
                   for j in range(nwin)]
        outs = []
        for j in range(nwin):
            gathers[j].wait()
            outs.append(pltpu.async_copy(
                row_vs[j], out_hbm.at[pl.ds((wid * nwin + j) * GW, GW)],
                sems[nwin + j]))
        for o in outs:
            o.wait()

    return k(data, idx_flat)


def _ffn_body(expv_ref, valid_ref, xg_ref, w1_ref, w2_ref, out_ref):
    c = pl.program_id(0)

    @pl.when(valid_ref[c] != 0)
    def _():
        h = jnp.dot(xg_ref[...], w1_ref[0], preferred_element_type=jnp.float32)
        a = jax.nn.gelu(h)
        out_ref[...] = jnp.dot(a.astype(jnp.bfloat16), w2_ref[0],
                               preferred_element_type=jnp.float32
                               ).astype(jnp.bfloat16)


def _ffn(expv, valid, xg, w1b, w2b):
    grid_spec = pltpu.PrefetchScalarGridSpec(
        num_scalar_prefetch=2,
        grid=(NCHUNK,),
        in_specs=[
            pl.BlockSpec((CHUNK, H), lambda c, expv, valid: (c, 0)),
            pl.BlockSpec((1, H, F), lambda c, expv, valid: (expv[c], 0, 0)),
            pl.BlockSpec((1, F, H), lambda c, expv, valid: (expv[c], 0, 0)),
        ],
        out_specs=pl.BlockSpec((CHUNK, H), lambda c, expv, valid: (c, 0)),
    )
    return pl.pallas_call(
        _ffn_body,
        grid_spec=grid_spec,
        out_shape=jax.ShapeDtypeStruct((NSLOT, H), jnp.bfloat16),
        compiler_params=pltpu.CompilerParams(
            dimension_semantics=("arbitrary",),
        ),
    )(expv, valid, xg, w1b, w2b)


def _combine_body(g_ref, p_ref, y_ref):
    y_ref[...] = (p_ref[:, 0:1] * g_ref[:, 0, :].astype(jnp.float32)
                  + p_ref[:, 1:2] * g_ref[:, 1, :].astype(jnp.float32))


def _combine(g3, probs):
    return pl.pallas_call(
        _combine_body,
        out_shape=jax.ShapeDtypeStruct((T, H), jnp.float32),
    )(g3, probs)


def _pack(a_bf16):
    n = a_bf16.shape[0]
    return lax.bitcast_convert_type(a_bf16.reshape(n, H // 2, 2), jnp.int32)


def _unpack(a_i32):
    n = a_i32.shape[0]
    return lax.bitcast_convert_type(a_i32, jnp.bfloat16).reshape(n, H)


@jax.jit
def kernel(x, Wr, W1, W2):
    pos, probs, expv, valid = _router(x, Wr)
    pos_flat = pos.reshape(NENT)
    expv_s = expv.reshape(NCPAD)[:NCHUNK]
    valid_s = valid.reshape(NCPAD)[:NCHUNK]

    gidx = _build_gidx(pos_flat, jnp.zeros((NSLOT,), jnp.int32))
    xg = _unpack(_sc_gather(_pack(x.astype(jnp.bfloat16)), gidx, NSLOT))

    w1b = W1.astype(jnp.bfloat16)
    w2b = W2.astype(jnp.bfloat16)
    o_cmp = _ffn(expv_s, valid_s, xg, w1b, w2b)

    g = _unpack(_sc_gather(_pack(o_cmp), pos_flat, NENT))
    return _combine(g.reshape(T, K, H), probs)


# speedup vs baseline: 4.2819x; 4.2819x over previous
import dataclasses
import functools

import jax
import jax.numpy as jnp
from jax import lax
from jax.experimental import pallas as pl
from jax.experimental.pallas import tpu as pltpu
from jax.experimental.pallas import tpu_sc as plsc

T = 2048
H = 1024
F = 4096
E = 8
K = 2

NENT = T * K
CHUNK = 256
NCHUNK = NENT // CHUNK + E
NSLOT = NCHUNK * CHUNK
NCPAD = 32

NWORK = 32
GW = 64
HW = H // 2


def _pack_f32(a):
    au = lax.bitcast_convert_type(a.astype(jnp.bfloat16).astype(jnp.float32),
                                  jnp.uint32)
    w = au[:, HW:] | jnp.right_shift(au[:, :HW], jnp.uint32(16))
    return lax.bitcast_convert_type(w, jnp.int32)


def _unpack_f32(w):
    wu = lax.bitcast_convert_type(w, jnp.uint32)
    lo = lax.bitcast_convert_type(jnp.left_shift(wu, jnp.uint32(16)),
                                  jnp.float32)
    hi = lax.bitcast_convert_type(wu & jnp.uint32(0xFFFF0000), jnp.float32)
    return jnp.concatenate([lo, hi], axis=1)


def _router_body(x_ref, wr_ref, pos_ref, prob_ref, expv_ref, valid_ref,
                 xpk_ref):
    xpk_ref[...] = _pack_f32(x_ref[...])
    logits = jnp.dot(x_ref[...], wr_ref[...], preferred_element_type=jnp.float32)
    eidx = lax.broadcasted_iota(jnp.int32, (T, E), 1)
    m1 = jnp.max(logits, axis=1, keepdims=True)
    i1 = jnp.min(jnp.where(logits == m1, eidx, E), axis=1, keepdims=True)
    l2 = jnp.where(eidx == i1, -jnp.inf, logits)
    m2 = jnp.max(l2, axis=1, keepdims=True)
    i2 = jnp.min(jnp.where(l2 == m2, eidx, E), axis=1, keepdims=True)
    e2 = jnp.exp(m2 - m1)
    p1 = 1.0 / (1.0 + e2)
    p2 = e2 / (1.0 + e2)

    a0 = jnp.where(eidx == i1, 1.0, 0.0)
    a1 = jnp.where(eidx == i2, 1.0, 0.0)
    b = a0 + a1

    cs_blk = 256
    r_iota = lax.broadcasted_iota(jnp.int32, (cs_blk, cs_blk), 0)
    c_iota = lax.broadcasted_iota(jnp.int32, (cs_blk, cs_blk), 1)
    ltri = jnp.where(r_iota > c_iota, 1.0, 0.0).astype(jnp.bfloat16)
    s_parts = []
    off = jnp.zeros((1, E), jnp.float32)
    for i in range(T // cs_blk):
        bi = b[i * cs_blk:(i + 1) * cs_blk]
        si = jnp.dot(ltri, bi.astype(jnp.bfloat16),
                     preferred_element_type=jnp.float32)
        s_parts.append(si + off)
        off = off + jnp.sum(bi, axis=0, keepdims=True)
    s = jnp.concatenate(s_parts, axis=0)

    counts = jnp.sum(b, axis=0, keepdims=True)
    nch = jnp.floor((counts + (CHUNK - 1)) * (1.0 / CHUNK))
    ru = lax.broadcasted_iota(jnp.int32, (E, E), 0)
    cu = lax.broadcasted_iota(jnp.int32, (E, E), 1)
    utri = jnp.where(ru < cu, 1.0, 0.0).astype(jnp.bfloat16)
    choff = jnp.dot(nch.astype(jnp.bfloat16), utri,
                    preferred_element_type=jnp.float32)
    slotbase = choff * float(CHUNK)

    base_bc = jnp.broadcast_to(slotbase, (T, E))
    pos0 = jnp.sum(a0 * (base_bc + s), axis=1, keepdims=True)
    pos1 = jnp.sum(a1 * (base_bc + s), axis=1, keepdims=True)

    pos_ref[...] = jnp.concatenate([pos0, pos1], axis=1).astype(jnp.int32)
    prob_ref[...] = jnp.concatenate([p1, p2], axis=1)

    total = jnp.sum(nch, axis=1, keepdims=True)
    cidx = lax.broadcasted_iota(jnp.int32, (NCPAD, E), 0).astype(jnp.float32)
    off_bc = jnp.broadcast_to(choff, (NCPAD, E))
    expv = jnp.sum(jnp.where(cidx >= off_bc, 1.0, 0.0), axis=1,
                   keepdims=True) - 1.0
    expv = jnp.clip(expv, 0.0, float(E - 1))
    expv_ref[...] = expv.astype(jnp.int32)
    cidx1 = lax.broadcasted_iota(jnp.int32, (NCPAD, 1), 0).astype(jnp.float32)
    valid_ref[...] = (cidx1 < jnp.broadcast_to(total, (NCPAD, 1))).astype(jnp.int32)


def _router(x, Wr):
    return pl.pallas_call(
        _router_body,
        out_shape=(
            jax.ShapeDtypeStruct((T, K), jnp.int32),
            jax.ShapeDtypeStruct((T, K), jnp.float32),
            jax.ShapeDtypeStruct((NCPAD, 1), jnp.int32),
            jax.ShapeDtypeStruct((NCPAD, 1), jnp.int32),
            jax.ShapeDtypeStruct((T, HW), jnp.int32),
        ),
    )(x, Wr)


def _sc_mesh():
    return plsc.VectorSubcoreMesh(core_axis_name="c", subcore_axis_name="s")


def _sc_compiler_params():
    cp = pltpu.CompilerParams()
    if "needs_layout_passes" in pltpu.CompilerParams.__dataclass_fields__:
        cp = dataclasses.replace(cp, needs_layout_passes=False)
    return cp


def _build_gidx(pos_flat, zeros_slot):
    @functools.partial(
        pl.kernel,
        mesh=_sc_mesh(),
        out_type=jax.ShapeDtypeStruct((NSLOT,), jnp.int32),
        compiler_params=_sc_compiler_params(),
        scratch_types=[
            pltpu.VMEM((NSLOT,), jnp.int32),
            pltpu.VMEM((NENT,), jnp.int32),
        ],
    )
    def k(pos_hbm, zeros_hbm, gidx_hbm, gidx_v, pos_v):
        wid = lax.axis_index("s") * 2 + lax.axis_index("c")

        @pl.when(wid == 0)
        def _():
            pltpu.sync_copy(zeros_hbm, gidx_v)
            pltpu.sync_copy(pos_hbm, pos_v)

            @pl.loop(0, NENT // 16)
            def _(i):
                idx = pos_v[pl.ds(i * 16, 16)]
                vals = lax.shift_right_logical(
                    lax.iota(jnp.int32, 16) + i * 16, 1)
                plsc.store_scatter(gidx_v, [idx], vals)

            pltpu.sync_copy(gidx_v, gidx_hbm)

    return k(pos_flat, zeros_slot)


def _sc_gather(data, idx_flat, nrows):
    w = data.shape[1]
    nwin = nrows // (NWORK * GW)

    @functools.partial(
        pl.kernel,
        mesh=_sc_mesh(),
        out_type=jax.ShapeDtypeStruct((nrows, w), jnp.int32),
        scratch_types=(
            [pltpu.VMEM((GW,), jnp.int32) for _ in range(nwin)]
            + [pltpu.VMEM((GW, w), jnp.int32) for _ in range(nwin)]
            + [pltpu.SemaphoreType.DMA for _ in range(2 * nwin)]
        ),
    )
    def k(data_hbm, idx_hbm, out_hbm, *refs):
        idx_vs = refs[:nwin]
        row_vs = refs[nwin:2 * nwin]
        sems = refs[2 * nwin:]
        wid = lax.axis_index("s") * 2 + lax.axis_index("c")

        for j in range(nwin):
            pltpu.sync_copy(idx_hbm.at[pl.ds((wid * nwin + j) * GW, GW)],
                            idx_vs[j])
        gathers = [pltpu.async_copy(data_hbm.at[idx_vs[j]], row_vs[j], sems[j])
                   for j in range(nwin)]
        outs = []
        for j in range(nwin):
            gathers[j].wait()
            outs.append(pltpu.async_copy(
                row_vs[j], out_hbm.at[pl.ds((wid * nwin + j) * GW, GW)],
                sems[nwin + j]))
        for o in outs:
            o.wait()

    return k(data, idx_flat)


def _ffn_body(expv_ref, valid_ref, xg_ref, w1_ref, w2_ref, out_ref):
    c = pl.program_id(0)

    @pl.when(valid_ref[c] != 0)
    def _():
        xa = _unpack_f32(xg_ref[...]).astype(jnp.bfloat16)
        h = jnp.dot(xa, w1_ref[0], preferred_element_type=jnp.float32)
        a = jax.nn.gelu(h)
        o = jnp.dot(a.astype(jnp.bfloat16), w2_ref[0],
                    preferred_element_type=jnp.float32)
        out_ref[...] = _pack_f32(o)


def _ffn(expv, valid, xg, w1b, w2b):
    grid_spec = pltpu.PrefetchScalarGridSpec(
        num_scalar_prefetch=2,
        grid=(NCHUNK,),
        in_specs=[
            pl.BlockSpec((CHUNK, HW), lambda c, expv, valid: (c, 0)),
            pl.BlockSpec((1, H, F), lambda c, expv, valid: (expv[c], 0, 0)),
            pl.BlockSpec((1, F, H), lambda c, expv, valid: (expv[c], 0, 0)),
        ],
        out_specs=pl.BlockSpec((CHUNK, HW), lambda c, expv, valid: (c, 0)),
    )
    return pl.pallas_call(
        _ffn_body,
        grid_spec=grid_spec,
        out_shape=jax.ShapeDtypeStruct((NSLOT, HW), jnp.int32),
        compiler_params=pltpu.CompilerParams(
            dimension_semantics=("arbitrary",),
        ),
    )(expv, valid, xg, w1b, w2b)


def _combine_body(g_ref, p_ref, y_ref):
    g0 = _unpack_f32(g_ref[:, 0, :])
    g1 = _unpack_f32(g_ref[:, 1, :])
    y_ref[...] = p_ref[:, 0:1] * g0 + p_ref[:, 1:2] * g1


def _combine(g3, probs):
    tb = 256
    return pl.pallas_call(
        _combine_body,
        grid=(T // tb,),
        in_specs=[
            pl.BlockSpec((tb, K, HW), lambda i: (i, 0, 0)),
            pl.BlockSpec((tb, K), lambda i: (i, 0)),
        ],
        out_specs=pl.BlockSpec((tb, H), lambda i: (i, 0)),
        out_shape=jax.ShapeDtypeStruct((T, H), jnp.float32),
    )(g3, probs)


@jax.jit
def kernel(x, Wr, W1, W2):
    pos, probs, expv, valid, xpk = _router(x, Wr)
    pos_flat = pos.reshape(NENT)
    expv_s = expv.reshape(NCPAD)[:NCHUNK]
    valid_s = valid.reshape(NCPAD)[:NCHUNK]

    gidx = _build_gidx(pos_flat, jnp.zeros((NSLOT,), jnp.int32))
    xg = _sc_gather(xpk, gidx, NSLOT)

    w1b = W1.astype(jnp.bfloat16)
    w2b = W2.astype(jnp.bfloat16)
    o_cmp = _ffn(expv_s, valid_s, xg, w1b, w2b)

    g = _sc_gather(o_cmp, pos_flat, NENT)
    return _combine(g.reshape(T, K, HW), probs)


# speedup vs baseline: 4.3278x; 1.0107x over previous
import dataclasses
import functools

import jax
import jax.numpy as jnp
from jax import lax
from jax.experimental import pallas as pl
from jax.experimental.pallas import tpu as pltpu
from jax.experimental.pallas import tpu_sc as plsc

T = 2048
H = 1024
F = 4096
E = 8
K = 2

NENT = T * K
CHUNK = 256
NCHUNK = NENT // CHUNK + E
NSLOT = NCHUNK * CHUNK
NCPAD = 32

NWORK = 32
GW = 64
HW = H // 2


def _pack_f32(a):
    au = lax.bitcast_convert_type(a.astype(jnp.bfloat16).astype(jnp.float32),
                                  jnp.uint32)
    w = au[:, HW:] | jnp.right_shift(au[:, :HW], jnp.uint32(16))
    return lax.bitcast_convert_type(w, jnp.int32)


def _unpack_f32(w):
    wu = lax.bitcast_convert_type(w, jnp.uint32)
    lo = lax.bitcast_convert_type(jnp.left_shift(wu, jnp.uint32(16)),
                                  jnp.float32)
    hi = lax.bitcast_convert_type(wu & jnp.uint32(0xFFFF0000), jnp.float32)
    return jnp.concatenate([lo, hi], axis=1)


def _router_body(x_ref, wr_ref, pos_ref, prob_ref, expv_ref, valid_ref,
                 xpk_ref):
    xpk_ref[...] = _pack_f32(x_ref[...])
    logits = jnp.dot(x_ref[...], wr_ref[...], preferred_element_type=jnp.float32)
    eidx = lax.broadcasted_iota(jnp.int32, (T, E), 1)
    m1 = jnp.max(logits, axis=1, keepdims=True)
    i1 = jnp.min(jnp.where(logits == m1, eidx, E), axis=1, keepdims=True)
    l2 = jnp.where(eidx == i1, -jnp.inf, logits)
    m2 = jnp.max(l2, axis=1, keepdims=True)
    i2 = jnp.min(jnp.where(l2 == m2, eidx, E), axis=1, keepdims=True)
    e2 = jnp.exp(m2 - m1)
    p1 = 1.0 / (1.0 + e2)
    p2 = e2 / (1.0 + e2)

    a0 = jnp.where(eidx == i1, 1.0, 0.0)
    a1 = jnp.where(eidx == i2, 1.0, 0.0)
    b = a0 + a1

    cs_blk = 256
    r_iota = lax.broadcasted_iota(jnp.int32, (cs_blk, cs_blk), 0)
    c_iota = lax.broadcasted_iota(jnp.int32, (cs_blk, cs_blk), 1)
    ltri = jnp.where(r_iota > c_iota, 1.0, 0.0).astype(jnp.bfloat16)
    s_parts = []
    off = jnp.zeros((1, E), jnp.float32)
    for i in range(T // cs_blk):
        bi = b[i * cs_blk:(i + 1) * cs_blk]
        si = jnp.dot(ltri, bi.astype(jnp.bfloat16),
                     preferred_element_type=jnp.float32)
        s_parts.append(si + off)
        off = off + jnp.sum(bi, axis=0, keepdims=True)
    s = jnp.concatenate(s_parts, axis=0)

    counts = jnp.sum(b, axis=0, keepdims=True)
    nch = jnp.floor((counts + (CHUNK - 1)) * (1.0 / CHUNK))
    ru = lax.broadcasted_iota(jnp.int32, (E, E), 0)
    cu = lax.broadcasted_iota(jnp.int32, (E, E), 1)
    utri = jnp.where(ru < cu, 1.0, 0.0).astype(jnp.bfloat16)
    choff = jnp.dot(nch.astype(jnp.bfloat16), utri,
                    preferred_element_type=jnp.float32)
    slotbase = choff * float(CHUNK)

    base_bc = jnp.broadcast_to(slotbase, (T, E))
    pos0 = jnp.sum(a0 * (base_bc + s), axis=1, keepdims=True)
    pos1 = jnp.sum(a1 * (base_bc + s), axis=1, keepdims=True)

    pos_ref[...] = jnp.concatenate([pos0, pos1], axis=1).astype(jnp.int32)
    prob_ref[...] = jnp.concatenate([p1, p2], axis=1)

    total = jnp.sum(nch, axis=1, keepdims=True)
    cidx = lax.broadcasted_iota(jnp.int32, (NCPAD, E), 0).astype(jnp.float32)
    off_bc = jnp.broadcast_to(choff, (NCPAD, E))
    expv = jnp.sum(jnp.where(cidx >= off_bc, 1.0, 0.0), axis=1,
                   keepdims=True) - 1.0
    expv = jnp.clip(expv, 0.0, float(E - 1))
    expv_ref[...] = expv.astype(jnp.int32)
    cidx1 = lax.broadcasted_iota(jnp.int32, (NCPAD, 1), 0).astype(jnp.float32)
    valid_ref[...] = (cidx1 < jnp.broadcast_to(total, (NCPAD, 1))).astype(jnp.int32)


def _router(x, Wr):
    return pl.pallas_call(
        _router_body,
        out_shape=(
            jax.ShapeDtypeStruct((T, K), jnp.int32),
            jax.ShapeDtypeStruct((T, K), jnp.float32),
            jax.ShapeDtypeStruct((NCPAD, 1), jnp.int32),
            jax.ShapeDtypeStruct((NCPAD, 1), jnp.int32),
            jax.ShapeDtypeStruct((T, HW), jnp.int32),
        ),
    )(x, Wr)


def _sc_mesh():
    return plsc.VectorSubcoreMesh(core_axis_name="c", subcore_axis_name="s")


def _sc_compiler_params():
    cp = pltpu.CompilerParams()
    if "needs_layout_passes" in pltpu.CompilerParams.__dataclass_fields__:
        cp = dataclasses.replace(cp, needs_layout_passes=False)
    return cp


def _dispatch_gather(pos_flat, zeros_slot, xpk):
    nwin = NSLOT // (NWORK * GW)

    @functools.partial(
        pl.kernel,
        mesh=_sc_mesh(),
        out_type=jax.ShapeDtypeStruct((NSLOT, HW), jnp.int32),
        compiler_params=_sc_compiler_params(),
        scratch_types=(
            [
                pltpu.VMEM((NSLOT,), jnp.int32),
                pltpu.VMEM((NENT,), jnp.int32),
                pltpu.VMEM_SHARED((NSLOT,), jnp.int32),
            ]
            + [pltpu.VMEM((GW,), jnp.int32) for _ in range(nwin)]
            + [pltpu.VMEM((GW, HW), jnp.int32) for _ in range(nwin)]
            + [pltpu.SemaphoreType.DMA for _ in range(2 * nwin)]
        ),
    )
    def k(pos_hbm, zeros_hbm, xpk_hbm, xg_hbm, gidx_v, pos_v, gidx_sh, *refs):
        idx_vs = refs[:nwin]
        row_vs = refs[nwin:2 * nwin]
        sems = refs[2 * nwin:]
        cid = lax.axis_index("c")
        sid = lax.axis_index("s")
        wid = sid * 2 + cid

        @pl.when(sid == 0)
        def _():
            pltpu.sync_copy(zeros_hbm, gidx_v)
            pltpu.sync_copy(pos_hbm, pos_v)

            @pl.loop(0, NENT // 16)
            def _(i):
                idx = pos_v[pl.ds(i * 16, 16)]
                vals = lax.shift_right_logical(
                    lax.iota(jnp.int32, 16) + i * 16, 1)
                plsc.store_scatter(gidx_v, [idx], vals)

            pltpu.sync_copy(gidx_v, gidx_sh)

        plsc.subcore_barrier()

        for j in range(nwin):
            pltpu.sync_copy(gidx_sh.at[pl.ds((wid * nwin + j) * GW, GW)],
                            idx_vs[j])
        gathers = [pltpu.async_copy(xpk_hbm.at[idx_vs[j]], row_vs[j], sems[j])
                   for j in range(nwin)]
        outs = []
        for j in range(nwin):
            gathers[j].wait()
            outs.append(pltpu.async_copy(
                row_vs[j], xg_hbm.at[pl.ds((wid * nwin + j) * GW, GW)],
                sems[nwin + j]))
        for o in outs:
            o.wait()

    return k(pos_flat, zeros_slot, xpk)


def _sc_gather(data, idx_flat, nrows):
    w = data.shape[1]
    nwin = nrows // (NWORK * GW)

    @functools.partial(
        pl.kernel,
        mesh=_sc_mesh(),
        out_type=jax.ShapeDtypeStruct((nrows, w), jnp.int32),
        scratch_types=(
            [pltpu.VMEM((GW,), jnp.int32) for _ in range(nwin)]
            + [pltpu.VMEM((GW, w), jnp.int32) for _ in range(nwin)]
            + [pltpu.SemaphoreType.DMA for _ in range(2 * nwin)]
        ),
    )
    def k(data_hbm, idx_hbm, out_hbm, *refs):
        idx_vs = refs[:nwin]
        row_vs = refs[nwin:2 * nwin]
        sems = refs[2 * nwin:]
        wid = lax.axis_index("s") * 2 + lax.axis_index("c")

        for j in range(nwin):
            pltpu.sync_copy(idx_hbm.at[pl.ds((wid * nwin + j) * GW, GW)],
                            idx_vs[j])
        gathers = [pltpu.async_copy(data_hbm.at[idx_vs[j]], row_vs[j], sems[j])
                   for j in range(nwin)]
        outs = []
        for j in range(nwin):
            gathers[j].wait()
            outs.append(pltpu.async_copy(
                row_vs[j], out_hbm.at[pl.ds((wid * nwin + j) * GW, GW)],
                sems[nwin + j]))
        for o in outs:
            o.wait()

    return k(data, idx_flat)


def _ffn_body(expv_ref, valid_ref, xg_ref, w1_ref, w2_ref, out_ref):
    c = pl.program_id(0)

    @pl.when(valid_ref[c] != 0)
    def _():
        xa = _unpack_f32(xg_ref[...]).astype(jnp.bfloat16)
        h = jnp.dot(xa, w1_ref[0], preferred_element_type=jnp.float32)
        a = jax.nn.gelu(h)
        o = jnp.dot(a.astype(jnp.bfloat16), w2_ref[0],
                    preferred_element_type=jnp.float32)
        out_ref[...] = _pack_f32(o)


def _ffn(expv, valid, xg, w1b, w2b):
    grid_spec = pltpu.PrefetchScalarGridSpec(
        num_scalar_prefetch=2,
        grid=(NCHUNK,),
        in_specs=[
            pl.BlockSpec((CHUNK, HW), lambda c, expv, valid: (c, 0)),
            pl.BlockSpec((1, H, F), lambda c, expv, valid: (expv[c], 0, 0)),
            pl.BlockSpec((1, F, H), lambda c, expv, valid: (expv[c], 0, 0)),
        ],
        out_specs=pl.BlockSpec((CHUNK, HW), lambda c, expv, valid: (c, 0)),
    )
    return pl.pallas_call(
        _ffn_body,
        grid_spec=grid_spec,
        out_shape=jax.ShapeDtypeStruct((NSLOT, HW), jnp.int32),
        compiler_params=pltpu.CompilerParams(
            dimension_semantics=("arbitrary",),
        ),
    )(expv, valid, xg, w1b, w2b)


def _combine_body(g_ref, p_ref, y_ref):
    g0 = _unpack_f32(g_ref[:, 0, :])
    g1 = _unpack_f32(g_ref[:, 1, :])
    y_ref[...] = p_ref[:, 0:1] * g0 + p_ref[:, 1:2] * g1


def _combine(g3, probs):
    tb = 256
    return pl.pallas_call(
        _combine_body,
        grid=(T // tb,),
        in_specs=[
            pl.BlockSpec((tb, K, HW), lambda i: (i, 0, 0)),
            pl.BlockSpec((tb, K), lambda i: (i, 0)),
        ],
        out_specs=pl.BlockSpec((tb, H), lambda i: (i, 0)),
        out_shape=jax.ShapeDtypeStruct((T, H), jnp.float32),
    )(g3, probs)


@jax.jit
def kernel(x, Wr, W1, W2):
    pos, probs, expv, valid, xpk = _router(x, Wr)
    pos_flat = pos.reshape(NENT)
    expv_s = expv.reshape(NCPAD)[:NCHUNK]
    valid_s = valid.reshape(NCPAD)[:NCHUNK]

    xg = _dispatch_gather(pos_flat, jnp.zeros((NSLOT,), jnp.int32), xpk)

    w1b = W1.astype(jnp.bfloat16)
    w2b = W2.astype(jnp.bfloat16)
    o_cmp = _ffn(expv_s, valid_s, xg, w1b, w2b)

    g = _sc_gather(o_cmp, pos_flat, NENT)
    return _combine(g.reshape(T, K, HW), probs)


# speedup vs baseline: 5.6591x; 1.3076x over previous
import dataclasses
import functools

import jax
import jax.numpy as jnp
from jax import lax
from jax.experimental import pallas as pl
from jax.experimental.pallas import tpu as pltpu
from jax.experimental.pallas import tpu_sc as plsc

T = 2048
H = 1024
F = 4096
E = 8
K = 2

NENT = T * K
CHUNK = 256
NCHUNK = NENT // CHUNK + E
NSLOT = NCHUNK * CHUNK
NCPAD = 32

NWORK = 32
GW = 64
HW = H // 2


def _pack_f32(a):
    au = lax.bitcast_convert_type(a.astype(jnp.bfloat16).astype(jnp.float32),
                                  jnp.uint32)
    w = au[:, HW:] | jnp.right_shift(au[:, :HW], jnp.uint32(16))
    return lax.bitcast_convert_type(w, jnp.int32)


def _unpack_f32(w):
    wu = lax.bitcast_convert_type(w, jnp.uint32)
    lo = lax.bitcast_convert_type(jnp.left_shift(wu, jnp.uint32(16)),
                                  jnp.float32)
    hi = lax.bitcast_convert_type(wu & jnp.uint32(0xFFFF0000), jnp.float32)
    return jnp.concatenate([lo, hi], axis=1)


def _router_body(x_ref, wr_ref, pos_ref, prob_ref, expv_ref, valid_ref,
                 xpk_ref):
    xpk_ref[...] = _pack_f32(x_ref[...])
    logits = jnp.dot(x_ref[...], wr_ref[...], preferred_element_type=jnp.float32)
    eidx = lax.broadcasted_iota(jnp.int32, (T, E), 1)
    m1 = jnp.max(logits, axis=1, keepdims=True)
    i1 = jnp.min(jnp.where(logits == m1, eidx, E), axis=1, keepdims=True)
    l2 = jnp.where(eidx == i1, -jnp.inf, logits)
    m2 = jnp.max(l2, axis=1, keepdims=True)
    i2 = jnp.min(jnp.where(l2 == m2, eidx, E), axis=1, keepdims=True)
    e2 = jnp.exp(m2 - m1)
    p1 = 1.0 / (1.0 + e2)
    p2 = e2 / (1.0 + e2)

    a0 = jnp.where(eidx == i1, 1.0, 0.0)
    a1 = jnp.where(eidx == i2, 1.0, 0.0)
    b = a0 + a1

    cs_blk = 256
    r_iota = lax.broadcasted_iota(jnp.int32, (cs_blk, cs_blk), 0)
    c_iota = lax.broadcasted_iota(jnp.int32, (cs_blk, cs_blk), 1)
    ltri = jnp.where(r_iota > c_iota, 1.0, 0.0).astype(jnp.bfloat16)
    s_parts = []
    off = jnp.zeros((1, E), jnp.float32)
    for i in range(T // cs_blk):
        bi = b[i * cs_blk:(i + 1) * cs_blk]
        si = jnp.dot(ltri, bi.astype(jnp.bfloat16),
                     preferred_element_type=jnp.float32)
        s_parts.append(si + off)
        off = off + jnp.sum(bi, axis=0, keepdims=True)
    s = jnp.concatenate(s_parts, axis=0)

    counts = jnp.sum(b, axis=0, keepdims=True)
    nch = jnp.floor((counts + (CHUNK - 1)) * (1.0 / CHUNK))
    ru = lax.broadcasted_iota(jnp.int32, (E, E), 0)
    cu = lax.broadcasted_iota(jnp.int32, (E, E), 1)
    utri = jnp.where(ru < cu, 1.0, 0.0).astype(jnp.bfloat16)
    choff = jnp.dot(nch.astype(jnp.bfloat16), utri,
                    preferred_element_type=jnp.float32)
    slotbase = choff * float(CHUNK)

    base_bc = jnp.broadcast_to(slotbase, (T, E))
    pos0 = jnp.sum(a0 * (base_bc + s), axis=1, keepdims=True)
    pos1 = jnp.sum(a1 * (base_bc + s), axis=1, keepdims=True)

    pos_ref[...] = jnp.concatenate([pos0, pos1], axis=1).astype(jnp.int32)
    prob_ref[...] = jnp.concatenate([p1, p2], axis=1)

    total = jnp.sum(nch, axis=1, keepdims=True)
    cidx = lax.broadcasted_iota(jnp.int32, (NCPAD, E), 0).astype(jnp.float32)
    off_bc = jnp.broadcast_to(choff, (NCPAD, E))
    expv = jnp.sum(jnp.where(cidx >= off_bc, 1.0, 0.0), axis=1,
                   keepdims=True) - 1.0
    expv = jnp.clip(expv, 0.0, float(E - 1))
    expv_ref[...] = expv.astype(jnp.int32)
    cidx1 = lax.broadcasted_iota(jnp.int32, (NCPAD, 1), 0).astype(jnp.float32)
    valid_ref[...] = (cidx1 < jnp.broadcast_to(total, (NCPAD, 1))).astype(jnp.int32)


def _router(x, Wr):
    return pl.pallas_call(
        _router_body,
        out_shape=(
            jax.ShapeDtypeStruct((T, K), jnp.int32),
            jax.ShapeDtypeStruct((T, K), jnp.float32),
            jax.ShapeDtypeStruct((NCPAD, 1), jnp.int32),
            jax.ShapeDtypeStruct((NCPAD, 1), jnp.int32),
            jax.ShapeDtypeStruct((T, HW), jnp.int32),
        ),
    )(x, Wr)


def _sc_mesh():
    return plsc.VectorSubcoreMesh(core_axis_name="c", subcore_axis_name="s")


def _sc_compiler_params():
    cp = pltpu.CompilerParams()
    if "needs_layout_passes" in pltpu.CompilerParams.__dataclass_fields__:
        cp = dataclasses.replace(cp, needs_layout_passes=False)
    return cp


def _dispatch_gather(pos_flat, zeros_slot, xpk):
    nwin = NSLOT // (NWORK * GW)

    @functools.partial(
        pl.kernel,
        mesh=_sc_mesh(),
        out_type=jax.ShapeDtypeStruct((NSLOT, HW), jnp.int32),
        compiler_params=_sc_compiler_params(),
        scratch_types=(
            [
                pltpu.VMEM((NSLOT,), jnp.int32),
                pltpu.VMEM((NENT,), jnp.int32),
                pltpu.VMEM_SHARED((NSLOT,), jnp.int32),
            ]
            + [pltpu.VMEM((GW,), jnp.int32) for _ in range(nwin)]
            + [pltpu.VMEM((GW, HW), jnp.int32) for _ in range(nwin)]
            + [pltpu.SemaphoreType.DMA for _ in range(2 * nwin)]
        ),
    )
    def k(pos_hbm, zeros_hbm, xpk_hbm, xg_hbm, gidx_v, pos_v, gidx_sh, *refs):
        idx_vs = refs[:nwin]
        row_vs = refs[nwin:2 * nwin]
        sems = refs[2 * nwin:]
        cid = lax.axis_index("c")
        sid = lax.axis_index("s")
        wid = sid * 2 + cid

        @pl.when(sid == 0)
        def _():
            pltpu.sync_copy(zeros_hbm, gidx_v)
            pltpu.sync_copy(pos_hbm, pos_v)

            @pl.loop(0, NENT // 16)
            def _(i):
                idx = pos_v[pl.ds(i * 16, 16)]
                vals = lax.shift_right_logical(
                    lax.iota(jnp.int32, 16) + i * 16, 1)
                plsc.store_scatter(gidx_v, [idx], vals)

            pltpu.sync_copy(gidx_v, gidx_sh)

        plsc.subcore_barrier()

        for j in range(nwin):
            pltpu.sync_copy(gidx_sh.at[pl.ds((wid * nwin + j) * GW, GW)],
                            idx_vs[j])
        gathers = [pltpu.async_copy(xpk_hbm.at[idx_vs[j]], row_vs[j], sems[j])
                   for j in range(nwin)]
        outs = []
        for j in range(nwin):
            gathers[j].wait()
            outs.append(pltpu.async_copy(
                row_vs[j], xg_hbm.at[pl.ds((wid * nwin + j) * GW, GW)],
                sems[nwin + j]))
        for o in outs:
            o.wait()

    return k(pos_flat, zeros_slot, xpk)


def _sc_gather(data, idx_flat, nrows):
    w = data.shape[1]
    nwin = nrows // (NWORK * GW)

    @functools.partial(
        pl.kernel,
        mesh=_sc_mesh(),
        out_type=jax.ShapeDtypeStruct((nrows, w), jnp.int32),
        scratch_types=(
            [pltpu.VMEM((GW,), jnp.int32) for _ in range(nwin)]
            + [pltpu.VMEM((GW, w), jnp.int32) for _ in range(nwin)]
            + [pltpu.SemaphoreType.DMA for _ in range(2 * nwin)]
        ),
    )
    def k(data_hbm, idx_hbm, out_hbm, *refs):
        idx_vs = refs[:nwin]
        row_vs = refs[nwin:2 * nwin]
        sems = refs[2 * nwin:]
        wid = lax.axis_index("s") * 2 + lax.axis_index("c")

        for j in range(nwin):
            pltpu.sync_copy(idx_hbm.at[pl.ds((wid * nwin + j) * GW, GW)],
                            idx_vs[j])
        gathers = [pltpu.async_copy(data_hbm.at[idx_vs[j]], row_vs[j], sems[j])
                   for j in range(nwin)]
        outs = []
        for j in range(nwin):
            gathers[j].wait()
            outs.append(pltpu.async_copy(
                row_vs[j], out_hbm.at[pl.ds((wid * nwin + j) * GW, GW)],
                sems[nwin + j]))
        for o in outs:
            o.wait()

    return k(data, idx_flat)


def _ffn_body(expv_ref, valid_ref, xg_ref, w1_ref, w2_ref, out_ref):
    c = pl.program_id(0)

    @pl.when(valid_ref[c] != 0)
    def _():
        xa = _unpack_f32(xg_ref[...]).astype(jnp.bfloat16)
        h = jnp.dot(xa, w1_ref[0], preferred_element_type=jnp.float32)
        a = jax.nn.gelu(h)
        o = jnp.dot(a.astype(jnp.bfloat16), w2_ref[0],
                    preferred_element_type=jnp.float32)
        out_ref[...] = _pack_f32(o)


def _ffn(expv, valid, xg, w1b, w2b):
    grid_spec = pltpu.PrefetchScalarGridSpec(
        num_scalar_prefetch=2,
        grid=(NCHUNK,),
        in_specs=[
            pl.BlockSpec((CHUNK, HW), lambda c, expv, valid: (c, 0)),
            pl.BlockSpec((1, H, F), lambda c, expv, valid: (expv[c], 0, 0)),
            pl.BlockSpec((1, F, H), lambda c, expv, valid: (expv[c], 0, 0)),
        ],
        out_specs=pl.BlockSpec((CHUNK, HW), lambda c, expv, valid: (c, 0)),
    )
    return pl.pallas_call(
        _ffn_body,
        grid_spec=grid_spec,
        out_shape=jax.ShapeDtypeStruct((NSLOT, HW), jnp.int32),
        compiler_params=pltpu.CompilerParams(
            dimension_semantics=("arbitrary",),
        ),
    )(expv, valid, xg, w1b, w2b)


def _combine_body(g_ref, p_ref, y_ref):
    g0 = _unpack_f32(g_ref[:, 0, :])
    g1 = _unpack_f32(g_ref[:, 1, :])
    y_ref[...] = p_ref[:, 0:1] * g0 + p_ref[:, 1:2] * g1


def _combine(g3, probs):
    tb = 256
    return pl.pallas_call(
        _combine_body,
        grid=(T // tb,),
        in_specs=[
            pl.BlockSpec((tb, K, HW), lambda i: (i, 0, 0)),
            pl.BlockSpec((tb, K), lambda i: (i, 0)),
        ],
        out_specs=pl.BlockSpec((tb, H), lambda i: (i, 0)),
        out_shape=jax.ShapeDtypeStruct((T, H), jnp.float32),
    )(g3, probs)


@jax.jit
def kernel(x, Wr, W1, W2):
    pos, probs, expv, valid, xpk = _router(x, Wr)
    pos_flat = pos.reshape(NENT)
    expv_s = expv.reshape(NCPAD)[:NCHUNK]
    valid_s = valid.reshape(NCPAD)[:NCHUNK]

    fill = jnp.arange(NSLOT, dtype=jnp.int32) & (T - 1)
    xg = _dispatch_gather(pos_flat, fill, xpk)

    w1b = W1.astype(jnp.bfloat16)
    w2b = W2.astype(jnp.bfloat16)
    o_cmp = _ffn(expv_s, valid_s, xg, w1b, w2b)

    g = _sc_gather(o_cmp, pos_flat, NENT)
    return _combine(g.reshape(T, K, HW), probs)


# speedup vs baseline: 5.6914x; 1.0057x over previous
import dataclasses
import functools

import jax
import jax.numpy as jnp
from jax import lax
from jax.experimental import pallas as pl
from jax.experimental.pallas import tpu as pltpu
from jax.experimental.pallas import tpu_sc as plsc

T = 2048
H = 1024
F = 4096
E = 8
K = 2

NENT = T * K
CHUNK = 256
NCHUNK = NENT // CHUNK + E
NSLOT = NCHUNK * CHUNK
NCPAD = 32

NWORK = 32
GW = 64
HW = H // 2


def _pack_f32(a):
    au = lax.bitcast_convert_type(a.astype(jnp.bfloat16).astype(jnp.float32),
                                  jnp.uint32)
    w = au[:, HW:] | jnp.right_shift(au[:, :HW], jnp.uint32(16))
    return lax.bitcast_convert_type(w, jnp.int32)


def _unpack_f32(w):
    wu = lax.bitcast_convert_type(w, jnp.uint32)
    lo = lax.bitcast_convert_type(jnp.left_shift(wu, jnp.uint32(16)),
                                  jnp.float32)
    hi = lax.bitcast_convert_type(wu & jnp.uint32(0xFFFF0000), jnp.float32)
    return jnp.concatenate([lo, hi], axis=1)


def _router_body(x_ref, wr_ref, pos_ref, prob_ref, expv_ref, valid_ref,
                 xpk_ref):
    xpk_ref[...] = _pack_f32(x_ref[...])
    logits = jnp.dot(x_ref[...], wr_ref[...], preferred_element_type=jnp.float32)
    eidx = lax.broadcasted_iota(jnp.int32, (T, E), 1)
    m1 = jnp.max(logits, axis=1, keepdims=True)
    i1 = jnp.min(jnp.where(logits == m1, eidx, E), axis=1, keepdims=True)
    l2 = jnp.where(eidx == i1, -jnp.inf, logits)
    m2 = jnp.max(l2, axis=1, keepdims=True)
    i2 = jnp.min(jnp.where(l2 == m2, eidx, E), axis=1, keepdims=True)
    e2 = jnp.exp(m2 - m1)
    p1 = 1.0 / (1.0 + e2)
    p2 = e2 / (1.0 + e2)

    a0 = jnp.where(eidx == i1, 1.0, 0.0)
    a1 = jnp.where(eidx == i2, 1.0, 0.0)
    b = a0 + a1

    cs_blk = 256
    r_iota = lax.broadcasted_iota(jnp.int32, (cs_blk, cs_blk), 0)
    c_iota = lax.broadcasted_iota(jnp.int32, (cs_blk, cs_blk), 1)
    ltri = jnp.where(r_iota > c_iota, 1.0, 0.0).astype(jnp.bfloat16)
    s_parts = []
    off = jnp.zeros((1, E), jnp.float32)
    for i in range(T // cs_blk):
        bi = b[i * cs_blk:(i + 1) * cs_blk]
        si = jnp.dot(ltri, bi.astype(jnp.bfloat16),
                     preferred_element_type=jnp.float32)
        s_parts.append(si + off)
        off = off + jnp.sum(bi, axis=0, keepdims=True)
    s = jnp.concatenate(s_parts, axis=0)

    counts = jnp.sum(b, axis=0, keepdims=True)
    nch = jnp.floor((counts + (CHUNK - 1)) * (1.0 / CHUNK))
    ru = lax.broadcasted_iota(jnp.int32, (E, E), 0)
    cu = lax.broadcasted_iota(jnp.int32, (E, E), 1)
    utri = jnp.where(ru < cu, 1.0, 0.0).astype(jnp.bfloat16)
    choff = jnp.dot(nch.astype(jnp.bfloat16), utri,
                    preferred_element_type=jnp.float32)
    slotbase = choff * float(CHUNK)

    base_bc = jnp.broadcast_to(slotbase, (T, E))
    pos0 = jnp.sum(a0 * (base_bc + s), axis=1, keepdims=True)
    pos1 = jnp.sum(a1 * (base_bc + s), axis=1, keepdims=True)

    pos_ref[...] = jnp.concatenate([pos0, pos1], axis=1).astype(jnp.int32)
    prob_ref[...] = jnp.concatenate([p1, p2], axis=1)

    total = jnp.sum(nch, axis=1, keepdims=True)
    cidx = lax.broadcasted_iota(jnp.int32, (NCPAD, E), 0).astype(jnp.float32)
    off_bc = jnp.broadcast_to(choff, (NCPAD, E))
    expv = jnp.sum(jnp.where(cidx >= off_bc, 1.0, 0.0), axis=1,
                   keepdims=True) - 1.0
    expv = jnp.clip(expv, 0.0, float(E - 1))
    expv_ref[...] = expv.astype(jnp.int32)
    cidx1 = lax.broadcasted_iota(jnp.int32, (NCPAD, 1), 0).astype(jnp.float32)
    valid_ref[...] = (cidx1 < jnp.broadcast_to(total, (NCPAD, 1))).astype(jnp.int32)


def _router(x, Wr):
    return pl.pallas_call(
        _router_body,
        out_shape=(
            jax.ShapeDtypeStruct((T, K), jnp.int32),
            jax.ShapeDtypeStruct((T, K), jnp.float32),
            jax.ShapeDtypeStruct((NCPAD, 1), jnp.int32),
            jax.ShapeDtypeStruct((NCPAD, 1), jnp.int32),
            jax.ShapeDtypeStruct((T, HW), jnp.int32),
        ),
    )(x, Wr)


def _sc_mesh():
    return plsc.VectorSubcoreMesh(core_axis_name="c", subcore_axis_name="s")


def _sc_compiler_params():
    cp = pltpu.CompilerParams()
    if "needs_layout_passes" in pltpu.CompilerParams.__dataclass_fields__:
        cp = dataclasses.replace(cp, needs_layout_passes=False)
    return cp


def _dispatch_gather(pos_flat, zeros_slot, xpk):
    nwin = NSLOT // (NWORK * GW)

    @functools.partial(
        pl.kernel,
        mesh=_sc_mesh(),
        out_type=jax.ShapeDtypeStruct((NSLOT, HW), jnp.int32),
        compiler_params=_sc_compiler_params(),
        scratch_types=(
            [
                pltpu.VMEM((NSLOT,), jnp.int32),
                pltpu.VMEM((NENT,), jnp.int32),
                pltpu.VMEM_SHARED((NSLOT,), jnp.int32),
            ]
            + [pltpu.VMEM((GW,), jnp.int32) for _ in range(nwin)]
            + [pltpu.VMEM((GW, HW), jnp.int32) for _ in range(nwin)]
            + [pltpu.SemaphoreType.DMA for _ in range(2 * nwin)]
        ),
    )
    def k(pos_hbm, zeros_hbm, xpk_hbm, xg_hbm, gidx_v, pos_v, gidx_sh, *refs):
        idx_vs = refs[:nwin]
        row_vs = refs[nwin:2 * nwin]
        sems = refs[2 * nwin:]
        cid = lax.axis_index("c")
        sid = lax.axis_index("s")
        wid = sid * 2 + cid

        @pl.when(sid == 0)
        def _():
            pltpu.sync_copy(zeros_hbm, gidx_v)
            pltpu.sync_copy(pos_hbm, pos_v)

            @pl.loop(0, NENT // 16)
            def _(i):
                idx = pos_v[pl.ds(i * 16, 16)]
                vals = lax.shift_right_logical(
                    lax.iota(jnp.int32, 16) + i * 16, 1)
                plsc.store_scatter(gidx_v, [idx], vals)

            pltpu.sync_copy(gidx_v, gidx_sh)

        plsc.subcore_barrier()

        for j in range(nwin):
            pltpu.sync_copy(gidx_sh.at[pl.ds((wid * nwin + j) * GW, GW)],
                            idx_vs[j])
        gathers = [pltpu.async_copy(xpk_hbm.at[idx_vs[j]], row_vs[j], sems[j])
                   for j in range(nwin)]
        outs = []
        for j in range(nwin):
            gathers[j].wait()
            outs.append(pltpu.async_copy(
                row_vs[j], xg_hbm.at[pl.ds((wid * nwin + j) * GW, GW)],
                sems[nwin + j]))
        for o in outs:
            o.wait()

    return k(pos_flat, zeros_slot, xpk)


def _sc_gather(data, idx_flat, nrows):
    w = data.shape[1]
    nwin = nrows // (NWORK * GW)

    @functools.partial(
        pl.kernel,
        mesh=_sc_mesh(),
        out_type=jax.ShapeDtypeStruct((nrows, w), jnp.int32),
        scratch_types=(
            [pltpu.VMEM((GW,), jnp.int32) for _ in range(nwin)]
            + [pltpu.VMEM((GW, w), jnp.int32) for _ in range(nwin)]
            + [pltpu.SemaphoreType.DMA for _ in range(2 * nwin)]
        ),
    )
    def k(data_hbm, idx_hbm, out_hbm, *refs):
        idx_vs = refs[:nwin]
        row_vs = refs[nwin:2 * nwin]
        sems = refs[2 * nwin:]
        wid = lax.axis_index("s") * 2 + lax.axis_index("c")

        for j in range(nwin):
            pltpu.sync_copy(idx_hbm.at[pl.ds((wid * nwin + j) * GW, GW)],
                            idx_vs[j])
        gathers = [pltpu.async_copy(data_hbm.at[idx_vs[j]], row_vs[j], sems[j])
                   for j in range(nwin)]
        outs = []
        for j in range(nwin):
            gathers[j].wait()
            outs.append(pltpu.async_copy(
                row_vs[j], out_hbm.at[pl.ds((wid * nwin + j) * GW, GW)],
                sems[nwin + j]))
        for o in outs:
            o.wait()

    return k(data, idx_flat)


def _ffn_body(expv_ref, valid_ref, xg_ref, w1_ref, w2_ref, out_ref):
    c = pl.program_id(0)

    @pl.when(valid_ref[c] != 0)
    def _():
        xa = _unpack_f32(xg_ref[...]).astype(jnp.bfloat16)
        h = jnp.dot(xa, w1_ref[0], preferred_element_type=jnp.float32)
        a = jax.nn.gelu(h.astype(jnp.bfloat16))
        o = jnp.dot(a, w2_ref[0], preferred_element_type=jnp.float32)
        out_ref[...] = _pack_f32(o)


def _ffn(expv, valid, xg, w1b, w2b):
    grid_spec = pltpu.PrefetchScalarGridSpec(
        num_scalar_prefetch=2,
        grid=(NCHUNK,),
        in_specs=[
            pl.BlockSpec((CHUNK, HW), lambda c, expv, valid: (c, 0)),
            pl.BlockSpec((1, H, F), lambda c, expv, valid: (expv[c], 0, 0)),
            pl.BlockSpec((1, F, H), lambda c, expv, valid: (expv[c], 0, 0)),
        ],
        out_specs=pl.BlockSpec((CHUNK, HW), lambda c, expv, valid: (c, 0)),
    )
    return pl.pallas_call(
        _ffn_body,
        grid_spec=grid_spec,
        out_shape=jax.ShapeDtypeStruct((NSLOT, HW), jnp.int32),
        compiler_params=pltpu.CompilerParams(
            dimension_semantics=("arbitrary",),
        ),
    )(expv, valid, xg, w1b, w2b)


def _combine_body(g_ref, p_ref, y_ref):
    g0 = _unpack_f32(g_ref[:, 0, :])
    g1 = _unpack_f32(g_ref[:, 1, :])
    y_ref[...] = p_ref[:, 0:1] * g0 + p_ref[:, 1:2] * g1


def _combine(g3, probs):
    tb = 512
    return pl.pallas_call(
        _combine_body,
        grid=(T // tb,),
        in_specs=[
            pl.BlockSpec((tb, K, HW), lambda i: (i, 0, 0)),
            pl.BlockSpec((tb, K), lambda i: (i, 0)),
        ],
        out_specs=pl.BlockSpec((tb, H), lambda i: (i, 0)),
        out_shape=jax.ShapeDtypeStruct((T, H), jnp.float32),
    )(g3, probs)


@jax.jit
def kernel(x, Wr, W1, W2):
    pos, probs, expv, valid, xpk = _router(x, Wr)
    pos_flat = pos.reshape(NENT)
    expv_s = expv.reshape(NCPAD)[:NCHUNK]
    valid_s = valid.reshape(NCPAD)[:NCHUNK]

    fill = jnp.arange(NSLOT, dtype=jnp.int32) & (T - 1)
    xg = _dispatch_gather(pos_flat, fill, xpk)

    w1b = W1.astype(jnp.bfloat16)
    w2b = W2.astype(jnp.bfloat16)
    o_cmp = _ffn(expv_s, valid_s, xg, w1b, w2b)

    g = _sc_gather(o_cmp, pos_flat, NENT)
    return _combine(g.reshape(T, K, HW), probs)


# speedup vs baseline: 5.7010x; 1.0017x over previous
import dataclasses
import functools

import jax
import jax.numpy as jnp
from jax import lax
from jax.experimental import pallas as pl
from jax.experimental.pallas import tpu as pltpu
from jax.experimental.pallas import tpu_sc as plsc

T = 2048
H = 1024
F = 4096
E = 8
K = 2

NENT = T * K
CHUNK = 256
NCHUNK = NENT // CHUNK + E
NSLOT = NCHUNK * CHUNK
NCPAD = 32

NWORK = 32
GW = 64
HW = H // 2


def _pack_f32(a):
    au = lax.bitcast_convert_type(a.astype(jnp.bfloat16).astype(jnp.float32),
                                  jnp.uint32)
    w = au[:, HW:] | jnp.right_shift(au[:, :HW], jnp.uint32(16))
    return lax.bitcast_convert_type(w, jnp.int32)


def _unpack_f32(w):
    wu = lax.bitcast_convert_type(w, jnp.uint32)
    lo = lax.bitcast_convert_type(jnp.left_shift(wu, jnp.uint32(16)),
                                  jnp.float32)
    hi = lax.bitcast_convert_type(wu & jnp.uint32(0xFFFF0000), jnp.float32)
    return jnp.concatenate([lo, hi], axis=1)


def _router_body(x_ref, wr_ref, pos_ref, prob_ref, expv_ref, valid_ref,
                 xpk_ref):
    xpk_ref[...] = _pack_f32(x_ref[...])
    logits = jnp.dot(x_ref[...], wr_ref[...], preferred_element_type=jnp.float32)
    eidx = lax.broadcasted_iota(jnp.int32, (T, E), 1)
    m1 = jnp.max(logits, axis=1, keepdims=True)
    i1 = jnp.min(jnp.where(logits == m1, eidx, E), axis=1, keepdims=True)
    l2 = jnp.where(eidx == i1, -jnp.inf, logits)
    m2 = jnp.max(l2, axis=1, keepdims=True)
    i2 = jnp.min(jnp.where(l2 == m2, eidx, E), axis=1, keepdims=True)
    e2 = jnp.exp(m2 - m1)
    p1 = 1.0 / (1.0 + e2)
    p2 = e2 / (1.0 + e2)

    a0 = jnp.where(eidx == i1, 1.0, 0.0)
    a1 = jnp.where(eidx == i2, 1.0, 0.0)
    b = a0 + a1

    cs_blk = 256
    r_iota = lax.broadcasted_iota(jnp.int32, (cs_blk, cs_blk), 0)
    c_iota = lax.broadcasted_iota(jnp.int32, (cs_blk, cs_blk), 1)
    ltri = jnp.where(r_iota > c_iota, 1.0, 0.0).astype(jnp.bfloat16)
    s_parts = []
    off = jnp.zeros((1, E), jnp.float32)
    for i in range(T // cs_blk):
        bi = b[i * cs_blk:(i + 1) * cs_blk]
        si = jnp.dot(ltri, bi.astype(jnp.bfloat16),
                     preferred_element_type=jnp.float32)
        s_parts.append(si + off)
        off = off + jnp.sum(bi, axis=0, keepdims=True)
    s = jnp.concatenate(s_parts, axis=0)

    counts = jnp.sum(b, axis=0, keepdims=True)
    nch = jnp.floor((counts + (CHUNK - 1)) * (1.0 / CHUNK))
    ru = lax.broadcasted_iota(jnp.int32, (E, E), 0)
    cu = lax.broadcasted_iota(jnp.int32, (E, E), 1)
    utri = jnp.where(ru < cu, 1.0, 0.0).astype(jnp.bfloat16)
    choff = jnp.dot(nch.astype(jnp.bfloat16), utri,
                    preferred_element_type=jnp.float32)
    slotbase = choff * float(CHUNK)

    base_bc = jnp.broadcast_to(slotbase, (T, E))
    pos0 = jnp.sum(a0 * (base_bc + s), axis=1, keepdims=True)
    pos1 = jnp.sum(a1 * (base_bc + s), axis=1, keepdims=True)

    pos_ref[...] = jnp.concatenate([pos0, pos1], axis=1).astype(jnp.int32)
    prob_ref[...] = jnp.concatenate([p1, p2], axis=1)

    total = jnp.sum(nch, axis=1, keepdims=True)
    cidx = lax.broadcasted_iota(jnp.int32, (NCPAD, E), 0).astype(jnp.float32)
    off_bc = jnp.broadcast_to(choff, (NCPAD, E))
    expv = jnp.sum(jnp.where(cidx >= off_bc, 1.0, 0.0), axis=1,
                   keepdims=True) - 1.0
    expv = jnp.clip(expv, 0.0, float(E - 1))
    expv_ref[...] = expv.astype(jnp.int32)
    cidx1 = lax.broadcasted_iota(jnp.int32, (NCPAD, 1), 0).astype(jnp.float32)
    valid_ref[...] = (cidx1 < jnp.broadcast_to(total, (NCPAD, 1))).astype(jnp.int32)


def _router(x, Wr):
    return pl.pallas_call(
        _router_body,
        out_shape=(
            jax.ShapeDtypeStruct((T, K), jnp.int32),
            jax.ShapeDtypeStruct((T, K), jnp.float32),
            jax.ShapeDtypeStruct((NCPAD, 1), jnp.int32),
            jax.ShapeDtypeStruct((NCPAD, 1), jnp.int32),
            jax.ShapeDtypeStruct((T, HW), jnp.int32),
        ),
    )(x, Wr)


def _sc_mesh():
    return plsc.VectorSubcoreMesh(core_axis_name="c", subcore_axis_name="s")


def _sc_compiler_params():
    cp = pltpu.CompilerParams()
    if "needs_layout_passes" in pltpu.CompilerParams.__dataclass_fields__:
        cp = dataclasses.replace(cp, needs_layout_passes=False)
    return cp


def _dispatch_gather(pos_flat, zeros_slot, xpk):
    nwin = NSLOT // (NWORK * GW)

    @functools.partial(
        pl.kernel,
        mesh=_sc_mesh(),
        out_type=jax.ShapeDtypeStruct((NSLOT, HW), jnp.int32),
        compiler_params=_sc_compiler_params(),
        scratch_types=(
            [
                pltpu.VMEM((NSLOT,), jnp.int32),
                pltpu.VMEM((NENT,), jnp.int32),
                pltpu.VMEM_SHARED((NSLOT,), jnp.int32),
            ]
            + [pltpu.VMEM((GW,), jnp.int32) for _ in range(nwin)]
            + [pltpu.VMEM((GW, HW), jnp.int32) for _ in range(nwin)]
            + [pltpu.SemaphoreType.DMA for _ in range(2 * nwin)]
        ),
    )
    def k(pos_hbm, zeros_hbm, xpk_hbm, xg_hbm, gidx_v, pos_v, gidx_sh, *refs):
        idx_vs = refs[:nwin]
        row_vs = refs[nwin:2 * nwin]
        sems = refs[2 * nwin:]
        cid = lax.axis_index("c")
        sid = lax.axis_index("s")
        wid = sid * 2 + cid

        @pl.when(sid == 0)
        def _():
            pltpu.sync_copy(zeros_hbm, gidx_v)
            pltpu.sync_copy(pos_hbm, pos_v)

            @pl.loop(0, NENT // 16)
            def _(i):
                idx = pos_v[pl.ds(i * 16, 16)]
                vals = lax.shift_right_logical(
                    lax.iota(jnp.int32, 16) + i * 16, 1)
                plsc.store_scatter(gidx_v, [idx], vals)

            pltpu.sync_copy(gidx_v, gidx_sh)

        plsc.subcore_barrier()

        for j in range(nwin):
            pltpu.sync_copy(gidx_sh.at[pl.ds((wid * nwin + j) * GW, GW)],
                            idx_vs[j])
        gathers = [pltpu.async_copy(xpk_hbm.at[idx_vs[j]], row_vs[j], sems[j])
                   for j in range(nwin)]
        outs = []
        for j in range(nwin):
            gathers[j].wait()
            outs.append(pltpu.async_copy(
                row_vs[j], xg_hbm.at[pl.ds((wid * nwin + j) * GW, GW)],
                sems[nwin + j]))
        for o in outs:
            o.wait()

    return k(pos_flat, zeros_slot, xpk)


def _sc_gather(data, idx_flat, nrows):
    w = data.shape[1]
    nwin = nrows // (NWORK * GW)

    @functools.partial(
        pl.kernel,
        mesh=_sc_mesh(),
        out_type=jax.ShapeDtypeStruct((nrows, w), jnp.int32),
        scratch_types=(
            [pltpu.VMEM((GW,), jnp.int32) for _ in range(nwin)]
            + [pltpu.VMEM((GW, w), jnp.int32) for _ in range(nwin)]
            + [pltpu.SemaphoreType.DMA for _ in range(2 * nwin)]
        ),
    )
    def k(data_hbm, idx_hbm, out_hbm, *refs):
        idx_vs = refs[:nwin]
        row_vs = refs[nwin:2 * nwin]
        sems = refs[2 * nwin:]
        wid = lax.axis_index("s") * 2 + lax.axis_index("c")

        for j in range(nwin):
            pltpu.sync_copy(idx_hbm.at[pl.ds((wid * nwin + j) * GW, GW)],
                            idx_vs[j])
        gathers = [pltpu.async_copy(data_hbm.at[idx_vs[j]], row_vs[j], sems[j])
                   for j in range(nwin)]
        outs = []
        for j in range(nwin):
            gathers[j].wait()
            outs.append(pltpu.async_copy(
                row_vs[j], out_hbm.at[pl.ds((wid * nwin + j) * GW, GW)],
                sems[nwin + j]))
        for o in outs:
            o.wait()

    return k(data, idx_flat)


def _ffn_body(expv_ref, valid_ref, xg_ref, w1_ref, w2_ref, out_ref):
    c = pl.program_id(0)

    @pl.when(valid_ref[c] != 0)
    def _():
        xa = _unpack_f32(xg_ref[...]).astype(jnp.bfloat16)
        fbb = 1024
        o = jnp.zeros((CHUNK, H), jnp.float32)
        a_prev = None
        for fb in range(F // fbb):
            h_fb = jnp.dot(xa, w1_ref[0, :, fb * fbb:(fb + 1) * fbb],
                           preferred_element_type=jnp.float32)
            if a_prev is not None:
                o = o + jnp.dot(a_prev, w2_ref[0, (fb - 1) * fbb:fb * fbb, :],
                                preferred_element_type=jnp.float32)
            a_prev = jax.nn.gelu(h_fb.astype(jnp.bfloat16))
        o = o + jnp.dot(a_prev, w2_ref[0, F - fbb:, :],
                        preferred_element_type=jnp.float32)
        out_ref[...] = _pack_f32(o)


def _ffn(expv, valid, xg, w1b, w2b):
    grid_spec = pltpu.PrefetchScalarGridSpec(
        num_scalar_prefetch=2,
        grid=(NCHUNK,),
        in_specs=[
            pl.BlockSpec((CHUNK, HW), lambda c, expv, valid: (c, 0)),
            pl.BlockSpec((1, H, F), lambda c, expv, valid: (expv[c], 0, 0)),
            pl.BlockSpec((1, F, H), lambda c, expv, valid: (expv[c], 0, 0)),
        ],
        out_specs=pl.BlockSpec((CHUNK, HW), lambda c, expv, valid: (c, 0)),
    )
    return pl.pallas_call(
        _ffn_body,
        grid_spec=grid_spec,
        out_shape=jax.ShapeDtypeStruct((NSLOT, HW), jnp.int32),
        compiler_params=pltpu.CompilerParams(
            dimension_semantics=("arbitrary",),
        ),
    )(expv, valid, xg, w1b, w2b)


def _combine_body(g_ref, p_ref, y_ref):
    g0 = _unpack_f32(g_ref[:, 0, :])
    g1 = _unpack_f32(g_ref[:, 1, :])
    y_ref[...] = p_ref[:, 0:1] * g0 + p_ref[:, 1:2] * g1


def _combine(g3, probs):
    tb = 512
    return pl.pallas_call(
        _combine_body,
        grid=(T // tb,),
        in_specs=[
            pl.BlockSpec((tb, K, HW), lambda i: (i, 0, 0)),
            pl.BlockSpec((tb, K), lambda i: (i, 0)),
        ],
        out_specs=pl.BlockSpec((tb, H), lambda i: (i, 0)),
        out_shape=jax.ShapeDtypeStruct((T, H), jnp.float32),
    )(g3, probs)


@jax.jit
def kernel(x, Wr, W1, W2):
    pos, probs, expv, valid, xpk = _router(x, Wr)
    pos_flat = pos.reshape(NENT)
    expv_s = expv.reshape(NCPAD)[:NCHUNK]
    valid_s = valid.reshape(NCPAD)[:NCHUNK]

    fill = jnp.arange(NSLOT, dtype=jnp.int32) & (T - 1)
    xg = _dispatch_gather(pos_flat, fill, xpk)

    w1b = W1.astype(jnp.bfloat16)
    w2b = W2.astype(jnp.bfloat16)
    o_cmp = _ffn(expv_s, valid_s, xg, w1b, w2b)

    g = _sc_gather(o_cmp, pos_flat, NENT)
    return _combine(g.reshape(T, K, HW), probs)
